# Initial kernel scaffold; baseline (speedup 1.0000x reference)
import jax
import jax.numpy as jnp
from jax.experimental import pallas as pl


def kernel(pos, batch, W1, b1, W2, b2, W3, b3):
    raise NotImplementedError("write your pallas kernel here")


# speedup vs baseline: 28.0942x; 28.0942x over previous
import functools

import jax
import jax.numpy as jnp
from jax import lax
from jax.experimental import pallas as pl
from jax.experimental.pallas import tpu as pltpu

N = 4096
NG = 8
KNN = 16
NPAD2 = 2304
RT = 256
BIGSLOT = 1 << 26
_INTERPRET = False


def _silu(x):
    return x * (1.0 / (1.0 + jnp.exp(-x)))


def _fps_body(sb_ref, px_ref, py_ref, pz_ref, bt_ref, slot_ref):
    g = pl.program_id(0)

    @pl.when(g == 0)
    def _():
        slot_ref[...] = jnp.full((8, N // 8), BIGSLOT, jnp.int32)

    start = sb_ref[0, g]
    mg = sb_ref[1, g]
    offg = sb_ref[2, g]

    @pl.when(mg > 0)
    def _():
        shp = (8, N // 8)
        gidx = (lax.broadcasted_iota(jnp.int32, shp, 0) * (N // 8)
                + lax.broadcasted_iota(jnp.int32, shp, 1))
        mask = bt_ref[...] == g
        x = px_ref[...]
        y = py_ref[...]
        z = pz_ref[...]

        oh0 = gidx == start
        xs = jnp.sum(jnp.where(oh0, x, 0.0))
        ys = jnp.sum(jnp.where(oh0, y, 0.0))
        zs = jnp.sum(jnp.where(oh0, z, 0.0))
        dx = x - xs
        dy = y - ys
        dz = z - zs
        d0 = jnp.where(mask, dx * dx + dy * dy + dz * dz, -1.0)
        slot_ref[...] = jnp.where(oh0, offg, slot_ref[...])

        def body(t, d):
            dmax = jnp.max(d)
            nxt = jnp.min(jnp.where(d == dmax, gidx, N))
            oh = gidx == nxt
            xn = jnp.sum(jnp.where(oh, x, 0.0))
            yn = jnp.sum(jnp.where(oh, y, 0.0))
            zn = jnp.sum(jnp.where(oh, z, 0.0))
            ex = x - xn
            ey = y - yn
            ez = z - zn
            dn = ex * ex + ey * ey + ez * ez
            d = jnp.where(mask, jnp.minimum(d, dn), -1.0)
            slot_ref[...] = jnp.where(oh, offg + t, slot_ref[...])
            return d

        lax.fori_loop(1, mg, body, d0)


def _run_fps(sbuf, px, py, pz, bt):
    return pl.pallas_call(
        _fps_body,
        grid=(NG,),
        in_specs=[
            pl.BlockSpec(memory_space=pltpu.SMEM),
            pl.BlockSpec((8, N // 8), lambda g: (0, 0)),
            pl.BlockSpec((8, N // 8), lambda g: (0, 0)),
            pl.BlockSpec((8, N // 8), lambda g: (0, 0)),
            pl.BlockSpec((8, N // 8), lambda g: (0, 0)),
        ],
        out_specs=pl.BlockSpec((8, N // 8), lambda g: (0, 0)),
        out_shape=jax.ShapeDtypeStruct((8, N // 8), jnp.int32),
        interpret=_INTERPRET,
    )(sbuf, px, py, pz, bt)


def _compact_body(slot_ref, p16_ref, out_ref):
    i = pl.program_id(0)
    rowid = i * RT + lax.broadcasted_iota(jnp.int32, (RT, 1), 0)
    oh = (slot_ref[...] == rowid).astype(jnp.float32)
    out_ref[...] = jnp.dot(oh, p16_ref[...], preferred_element_type=jnp.float32)


def _run_compact(slot_row, p16):
    return pl.pallas_call(
        _compact_body,
        grid=(NPAD2 // RT,),
        in_specs=[
            pl.BlockSpec((1, N), lambda i: (0, 0)),
            pl.BlockSpec((N, 16), lambda i: (0, 0)),
        ],
        out_specs=pl.BlockSpec((RT, 16), lambda i: (i, 0)),
        out_shape=jax.ShapeDtypeStruct((NPAD2, 16), jnp.float32),
        interpret=_INTERPRET,
    )(slot_row, p16)


def _knn_conv(d, a_full, ncols):
    citer = lax.broadcasted_iota(jnp.int32, d.shape, 1)
    m = jnp.full((d.shape[0], a_full.shape[1]), -jnp.inf, jnp.float32)
    for _ in range(KNN):
        cur = jnp.min(d, axis=1, keepdims=True)
        idx = jnp.min(jnp.where(d == cur, citer, ncols), axis=1, keepdims=True)
        sel = citer == idx
        oh = sel.astype(jnp.float32)
        gat = jnp.dot(oh, a_full, preferred_element_type=jnp.float32)
        m = jnp.maximum(m, gat)
        d = jnp.where(sel, jnp.inf, d)
    return m


def _stage1_body(p16c_ref, p8t_ref, p16_ref, btrow_ref, bc_ref,
                 w1sum_ref, w1b_ref, b1_ref, h_ref):
    rows = p16c_ref[...]
    rowm = jnp.concatenate(
        [rows[:, :3], jnp.zeros((RT, 5), jnp.float32)], axis=1)
    g = jnp.dot(rowm, p8t_ref[...], preferred_element_type=jnp.float32)
    sqi = rows[:, 3:4]
    sqj = p8t_ref[3:4, :]
    d = (sqi + sqj) - 2.0 * g
    d = jnp.where(btrow_ref[...] != bc_ref[...], jnp.inf, d)

    a1 = jnp.dot(p16_ref[...][:, :8], w1sum_ref[...],
                 preferred_element_type=jnp.float32)
    m = _knn_conv(d, a1, N)
    c = jnp.dot(rows[:, :8], w1b_ref[...],
                preferred_element_type=jnp.float32) - b1_ref[...]
    h_ref[...] = _silu(m - c)


def _run_stage1(p16c, p8t, p16, btrow, bc_col, w1sum8, w1b8, b1r):
    return pl.pallas_call(
        _stage1_body,
        grid=(NPAD2 // RT,),
        in_specs=[
            pl.BlockSpec((RT, 16), lambda i: (i, 0)),
            pl.BlockSpec((8, N), lambda i: (0, 0)),
            pl.BlockSpec((N, 16), lambda i: (0, 0)),
            pl.BlockSpec((1, N), lambda i: (0, 0)),
            pl.BlockSpec((RT, 1), lambda i: (i, 0)),
            pl.BlockSpec((8, 32), lambda i: (0, 0)),
            pl.BlockSpec((8, 32), lambda i: (0, 0)),
            pl.BlockSpec((1, 32), lambda i: (0, 0)),
        ],
        out_specs=pl.BlockSpec((RT, 32), lambda i: (i, 0)),
        out_shape=jax.ShapeDtypeStruct((NPAD2, 32), jnp.float32),
        interpret=_INTERPRET,
    )(p16c, p8t, p16, btrow, bc_col, w1sum8, w1b8, b1r)


def _stage2_body(p16c_ref, p2t_ref, h_ref, bcrow_ref, bc_ref, p16cfull_ref,
                 w2a_ref, w2b_ref, b2_ref, w3_ref, b3_ref, out_ref, gacc):
    i = pl.program_id(0)

    @pl.when(i == 0)
    def _():
        gacc[...] = jnp.full((8, 32), -jnp.inf, jnp.float32)

    rows = p16c_ref[...]
    rowm = jnp.concatenate(
        [rows[:, :3], jnp.zeros((RT, 13), jnp.float32)], axis=1)
    g = jnp.dot(rowm, p2t_ref[...], preferred_element_type=jnp.float32)
    sqi = rows[:, 3:4]
    sqj = p2t_ref[3:4, :]
    d = (sqi + sqj) - 2.0 * g
    d = jnp.where(bcrow_ref[...] != bc_ref[...], jnp.inf, d)

    a2 = (jnp.dot(h_ref[...], w2a_ref[...], preferred_element_type=jnp.float32)
          + jnp.dot(p16cfull_ref[...], w2b_ref[...],
                    preferred_element_type=jnp.float32))
    m = _knn_conv(d, a2, NPAD2)
    c = jnp.dot(rows, w2b_ref[...],
                preferred_element_type=jnp.float32) - b2_ref[...]
    h2 = _silu(m - c)

    bt = bc_ref[...]
    for gg in range(NG):
        red = jnp.max(jnp.where(bt == gg, h2, -jnp.inf), axis=0)
        gacc[gg, :] = jnp.maximum(gacc[gg, :], red)

    @pl.when(i == pl.num_programs(0) - 1)
    def _():
        out_ref[...] = (jnp.dot(gacc[...], w3_ref[...],
                                preferred_element_type=jnp.float32)
                        + b3_ref[...])


def kernel(pos, batch, W1, b1, W2, b2, W3, b3):
    pos = pos.astype(jnp.float32)
    batch = batch.astype(jnp.int32)
    sq = jnp.sum(pos * pos, axis=-1)

    starts = jnp.searchsorted(batch, jnp.arange(NG + 1, dtype=jnp.int32)
                              ).astype(jnp.int32)
    n = starts[1:] - starts[:-1]
    m = jnp.where(
        n > 0,
        jnp.maximum(1, jnp.ceil(0.5 * n.astype(jnp.float32)).astype(jnp.int32)),
        0,
    )
    off = jnp.concatenate([jnp.zeros(1, jnp.int32),
                           jnp.cumsum(m).astype(jnp.int32)])
    sbuf = jnp.zeros((3, 16), jnp.int32)
    sbuf = sbuf.at[0, :NG].set(starts[:NG])
    sbuf = sbuf.at[1, :NG].set(m)
    sbuf = sbuf.at[2, :NG].set(off[:NG])

    px = pos[:, 0].reshape(8, N // 8)
    py = pos[:, 1].reshape(8, N // 8)
    pz = pos[:, 2].reshape(8, N // 8)
    bt8 = batch.reshape(8, N // 8)

    slot = _run_fps(sbuf, px, py, pz, bt8)
    slot_row = slot.reshape(1, N)

    p16 = jnp.concatenate(
        [pos, sq[:, None], jnp.zeros((N, 12), jnp.float32)], axis=1)
    p16c = _run_compact(slot_row, p16)

    pidx = jnp.arange(NPAD2, dtype=jnp.int32)
    batch_c = jnp.sum(pidx[:, None] >= off[None, 1:], axis=1).astype(jnp.int32)

    p8t = p16[:, :8].T
    btrow = batch.reshape(1, N)
    bc_col = batch_c.reshape(NPAD2, 1)

    w1sum8 = jnp.concatenate([W1[:3] + W1[3:], jnp.zeros((5, 32))], axis=0)
    w1b8 = jnp.concatenate([W1[3:], jnp.zeros((5, 32))], axis=0)
    h = _run_stage1(p16c, p8t, p16, btrow, bc_col,
                    w1sum8.astype(jnp.float32), w1b8.astype(jnp.float32),
                    b1.reshape(1, 32).astype(jnp.float32))

    p2t = p16c.T
    bcrow = batch_c.reshape(1, NPAD2)
    w2a = W2[:32].astype(jnp.float32)
    w2b16 = jnp.concatenate([W2[32:], jnp.zeros((13, 32))], axis=0
                            ).astype(jnp.float32)

    out = pl.pallas_call(
        _stage2_body,
        grid=(NPAD2 // RT,),
        in_specs=[
            pl.BlockSpec((RT, 16), lambda i: (i, 0)),
            pl.BlockSpec((16, NPAD2), lambda i: (0, 0)),
            pl.BlockSpec((NPAD2, 32), lambda i: (0, 0)),
            pl.BlockSpec((1, NPAD2), lambda i: (0, 0)),
            pl.BlockSpec((RT, 1), lambda i: (i, 0)),
            pl.BlockSpec((NPAD2, 16), lambda i: (0, 0)),
            pl.BlockSpec((32, 32), lambda i: (0, 0)),
            pl.BlockSpec((16, 32), lambda i: (0, 0)),
            pl.BlockSpec((1, 32), lambda i: (0, 0)),
            pl.BlockSpec((32, 128), lambda i: (0, 0)),
            pl.BlockSpec((1, 128), lambda i: (0, 0)),
        ],
        out_specs=pl.BlockSpec((NG, 128), lambda i: (0, 0)),
        out_shape=jax.ShapeDtypeStruct((NG, 128), jnp.float32),
        scratch_shapes=[pltpu.VMEM((8, 32), jnp.float32)],
        interpret=_INTERPRET,
    )(p16c, p2t, h, bcrow, bc_col, p16c,
      w2a, w2b16, b2.reshape(1, 32).astype(jnp.float32),
      W3.astype(jnp.float32), b3.reshape(1, 128).astype(jnp.float32))
    return out
